# baseline (speedup 1.0000x reference)
import functools

import jax
import jax.numpy as jnp
from jax.experimental import pallas as pl
from jax.experimental.pallas import tpu as pltpu

_VMEM_LIMIT = 64 * 1024 * 1024
_L = 32
_TP = 256
_GU = 8


def _dist_knn_kernel(nsel, xr_ref, xt_ref, dist_ref, idx_ref):
    TI, N = dist_ref.shape
    d = jnp.zeros((TI, N), jnp.float32)
    for c in range(3):
        diff = xr_ref[:, c:c + 1] - xt_ref[c:c + 1, :]
        d = d + diff * diff
    d = jnp.where(d < 1e-8, jnp.inf, d)
    dist_ref[...] = d

    iota = jax.lax.broadcasted_iota(jnp.int32, (TI, N), 1)
    self_col = (jax.lax.broadcasted_iota(jnp.int32, (TI, 1), 0)
                + pl.program_id(1) * TI)
    cols = [self_col]
    w = d
    for _ in range(nsel):
        g = jnp.min(w, axis=1, keepdims=True)
        cand = jnp.where(w == g, iota, N)
        j = jnp.min(cand, axis=1, keepdims=True)
        cols.append(j)
        w = jnp.where(cand == j, jnp.inf, w)
    idx_ref[...] = jnp.concatenate(cols, axis=1)


def _masked_sqdist_knn(xyz, L):
    BS, N, _ = xyz.shape
    xyz = xyz.astype(jnp.float32)
    TI = 256 if N % 256 == 0 else N
    xyz_t = jnp.transpose(xyz, (0, 2, 1))
    return pl.pallas_call(
        functools.partial(_dist_knn_kernel, L - 1),
        out_shape=(jax.ShapeDtypeStruct((BS, N, N), jnp.float32),
                   jax.ShapeDtypeStruct((BS, N, L), jnp.int32)),
        grid=(BS, N // TI),
        in_specs=[pl.BlockSpec((None, TI, 3), lambda b, i: (b, i, 0)),
                  pl.BlockSpec((None, 3, N), lambda b, i: (b, 0, 0))],
        out_specs=(pl.BlockSpec((None, TI, N), lambda b, i: (b, i, 0)),
                   pl.BlockSpec((None, TI, L), lambda b, i: (b, i, 0))),
        compiler_params=pltpu.CompilerParams(
            dimension_semantics=("parallel", "arbitrary"),
            vmem_limit_bytes=_VMEM_LIMIT),
    )(xyz, xyz_t)


def _fused_kernel(idx_ref, src_ref, ds_ref, hs_ref, a0_ref, b1_ref, w2_ref,
                  b2_ref, wc_ref, bc_ref, out_ref, gd_ref, gh_ref):
    L, tp = idx_ref.shape
    H = w2_ref.shape[0]
    K = w2_ref.shape[1]
    C_out = out_ref.shape[1]

    def gather_chunk(c, carry):
        for u in range(_GU):
            p = c * _GU + u
            for l in range(1, L):
                i2 = pl.multiple_of(idx_ref[l, p], 2)
                slab = src_ref[pl.ds(i2, 2), :]
                slot = l * tp + p
                gd_ref[pl.ds(slot, 1), :] = slab[0:1, :]
                gh_ref[pl.ds(slot, 1), :] = slab[1:2, :]
        return carry

    jax.lax.fori_loop(0, tp // _GU, gather_chunk, 0, unroll=False)

    b1 = b1_ref[...]
    b2 = b2_ref[...]
    h0 = hs_ref[...]
    wc = wc_ref[...]

    w2 = w2_ref[...]

    def neighbor_body(l, accs):
        lo = pl.multiple_of(l * tp, tp)
        hg = gh_ref[pl.ds(lo, tp), 0:H]
        h = jnp.maximum(hg - h0 + b1, 0.0)
        lg = jnp.dot(h, w2, preferred_element_type=jnp.float32) + b2
        lg = lg - jnp.max(lg, axis=1, keepdims=True)
        e = jnp.exp(lg)
        a = e / jnp.sum(e, axis=1, keepdims=True)
        d = gd_ref[pl.ds(lo, tp), :]
        g = jnp.dot(d, wc, preferred_element_type=jnp.float32)
        return tuple(
            acc + sum(g[:, k * C_out:(k + 1) * C_out] * a[:, k:k + 1]
                      for k in range(t, K, 4))
            for t, acc in enumerate(accs))

    a0 = jnp.broadcast_to(a0_ref[...], (tp, K))
    g0 = jnp.dot(ds_ref[...], wc, preferred_element_type=jnp.float32)
    accs0 = tuple(
        (jnp.broadcast_to(bc_ref[...], (tp, C_out)) if t == 0 else
         jnp.zeros((tp, C_out), jnp.float32))
        + sum(g0[:, k * C_out:(k + 1) * C_out] * a0[:, k:k + 1]
              for k in range(t, K, 4))
        for t in range(4))
    accs = jax.lax.fori_loop(1, L, neighbor_body, accs0)
    out_ref[...] = (accs[0] + accs[1]) + (accs[2] + accs[3])


def kernel(xyz, data, w1f, b1f, w2t, b2c, wckc, bcc):
    BS, N, C = data.shape
    L = _L
    K, C_out = wckc.shape[0], wckc.shape[1]
    H = w1f.shape[0]
    tp = _TP if N % _TP == 0 else N

    dist, local_index = _masked_sqdist_knn(xyz, L)

    idx2 = jnp.transpose(local_index, (0, 2, 1)) * 2

    hx = jnp.einsum("bnd,hd->bnh", xyz.astype(jnp.float32), w1f)
    hx_pad = jnp.pad(hx, ((0, 0), (0, 0), (0, C - H)))
    src2d = jnp.concatenate(
        [data.astype(jnp.float32)[:, :, None, :], hx_pad[:, :, None, :]],
        axis=2).reshape(BS, 2 * N, C)

    b1r = jnp.transpose(b1f, (1, 0))
    w2T = jnp.transpose(w2t, (1, 0))
    b2r = jnp.transpose(b2c, (1, 0))
    wcT = jnp.transpose(wckc, (2, 0, 1)).reshape(C, K * C_out)
    bcr = jnp.transpose(bcc, (1, 0))

    lg0 = jnp.dot(jnp.maximum(b1r, 0.0), w2T) + b2r
    lg0 = lg0 - jnp.max(lg0, axis=1, keepdims=True)
    e0 = jnp.exp(lg0)
    a0 = e0 / jnp.sum(e0, axis=1, keepdims=True)

    full2 = lambda b, i: (0, 0)
    out = pl.pallas_call(
        _fused_kernel,
        out_shape=jax.ShapeDtypeStruct((BS, N, C_out), jnp.float32),
        grid=(BS, N // tp),
        in_specs=[
            pl.BlockSpec((None, L, tp), lambda b, i: (b, 0, i),
                         memory_space=pltpu.SMEM),
            pl.BlockSpec((None, 2 * N, C), lambda b, i: (b, 0, 0)),
            pl.BlockSpec((None, tp, C), lambda b, i: (b, i, 0)),
            pl.BlockSpec((None, tp, H), lambda b, i: (b, i, 0)),
            pl.BlockSpec((1, K), full2),
            pl.BlockSpec((1, H), full2),
            pl.BlockSpec((H, K), full2),
            pl.BlockSpec((1, K), full2),
            pl.BlockSpec((C, K * C_out), full2),
            pl.BlockSpec((1, C_out), full2),
        ],
        out_specs=pl.BlockSpec((None, tp, C_out), lambda b, i: (b, i, 0)),
        scratch_shapes=[pltpu.VMEM((L * tp, C), jnp.float32),
                        pltpu.VMEM((L * tp, C), jnp.float32)],
        compiler_params=pltpu.CompilerParams(
            dimension_semantics=("parallel", "arbitrary"),
            vmem_limit_bytes=_VMEM_LIMIT),
    )(idx2, src2d, data.astype(jnp.float32), hx, a0, b1r, w2T, b2r, wcT, bcr)

    aux = {"local_index": local_index, "dist": dist}
    return (xyz, out), aux


# speedup vs baseline: 1.1778x; 1.1778x over previous
import functools

import jax
import jax.numpy as jnp
from jax.experimental import pallas as pl
from jax.experimental.pallas import tpu as pltpu

_VMEM_LIMIT = 64 * 1024 * 1024
_L = 32
_TP = 256
_GU = 8


def _dist_knn_kernel(nsel, xr_ref, xt_ref, dist_ref, idx_ref):
    TI, N = dist_ref.shape
    d = jnp.zeros((TI, N), jnp.float32)
    for c in range(3):
        diff = xr_ref[:, c:c + 1] - xt_ref[c:c + 1, :]
        d = d + diff * diff
    d = jnp.where(d < 1e-8, jnp.inf, d)
    dist_ref[...] = d

    iota = jax.lax.broadcasted_iota(jnp.int32, (TI, N), 1)
    self_col = (jax.lax.broadcasted_iota(jnp.int32, (TI, 1), 0)
                + pl.program_id(1) * TI)
    cols = [self_col]
    w = d
    for _ in range(nsel):
        g = jnp.min(w, axis=1, keepdims=True)
        cand = jnp.where(w == g, iota, N)
        j = jnp.min(cand, axis=1, keepdims=True)
        cols.append(j)
        w = jnp.where(cand == j, jnp.inf, w)
    idx_ref[...] = jnp.concatenate(cols, axis=1)


def _masked_sqdist_knn(xyz, L):
    BS, N, _ = xyz.shape
    xyz = xyz.astype(jnp.float32)
    TI = 256 if N % 256 == 0 else N
    xyz_t = jnp.transpose(xyz, (0, 2, 1))
    return pl.pallas_call(
        functools.partial(_dist_knn_kernel, L - 1),
        out_shape=(jax.ShapeDtypeStruct((BS, N, N), jnp.float32),
                   jax.ShapeDtypeStruct((BS, N, L), jnp.int32)),
        grid=(BS, N // TI),
        in_specs=[pl.BlockSpec((None, TI, 3), lambda b, i: (b, i, 0)),
                  pl.BlockSpec((None, 3, N), lambda b, i: (b, 0, 0))],
        out_specs=(pl.BlockSpec((None, TI, N), lambda b, i: (b, i, 0)),
                   pl.BlockSpec((None, TI, L), lambda b, i: (b, i, 0))),
        compiler_params=pltpu.CompilerParams(
            dimension_semantics=("parallel", "arbitrary"),
            vmem_limit_bytes=_VMEM_LIMIT),
    )(xyz, xyz_t)


def _fused_kernel(idx_ref, src_ref, ds_ref, hs_ref, a0_ref, b1_ref, w2_ref,
                  b2_ref, wc_ref, bc_ref, out_ref, gd_ref, gh_ref, a_ref):
    L, tp = idx_ref.shape
    H = w2_ref.shape[0]
    K = w2_ref.shape[1]
    C_out = out_ref.shape[1]

    def gather_chunk(c, carry):
        for u in range(_GU):
            p = c * _GU + u
            for l in range(1, L):
                i2 = pl.multiple_of(idx_ref[l, p], 2)
                slab = src_ref[pl.ds(i2, 2), :]
                slot = l * tp + p
                gd_ref[pl.ds(slot, 1), :] = slab[0:1, :]
                gh_ref[pl.ds(slot, 1), :] = slab[1:2, :]
        return carry

    jax.lax.fori_loop(0, tp // _GU, gather_chunk, 0, unroll=False)

    b1 = b1_ref[...]
    b2 = b2_ref[...]
    h0 = hs_ref[...]

    for l in range(1, L):
        hg = gh_ref[l * tp:(l + 1) * tp, 0:H]
        h = jnp.maximum(hg - h0 + b1, 0.0)
        lg = jnp.dot(h, w2_ref[...],
                     preferred_element_type=jnp.float32) + b2
        lg = lg - jnp.max(lg, axis=1, keepdims=True)
        e = jnp.exp(lg)
        a_ref[l * tp:(l + 1) * tp, :] = e / jnp.sum(e, axis=1, keepdims=True)

    KT = 4
    dself = ds_ref[...]
    out = jnp.broadcast_to(bc_ref[...], (tp, C_out))
    for k0 in range(0, K, KT):
        accs = [dself * a0_ref[0:1, k0 + t:k0 + t + 1] for t in range(KT)]
        for l in range(1, L):
            d = gd_ref[l * tp:(l + 1) * tp, :]
            al = a_ref[l * tp:(l + 1) * tp, k0:k0 + KT]
            for t in range(KT):
                accs[t] = accs[t] + d * al[:, t:t + 1]
        for t in range(KT):
            out = out + jnp.dot(accs[t], wc_ref[k0 + t],
                                preferred_element_type=jnp.float32)
    out_ref[...] = out


def kernel(xyz, data, w1f, b1f, w2t, b2c, wckc, bcc):
    BS, N, C = data.shape
    L = _L
    K, C_out = wckc.shape[0], wckc.shape[1]
    H = w1f.shape[0]
    tp = _TP if N % _TP == 0 else N

    dist, local_index = _masked_sqdist_knn(xyz, L)

    idx2 = jnp.transpose(local_index, (0, 2, 1)) * 2

    hx = jnp.einsum("bnd,hd->bnh", xyz.astype(jnp.float32), w1f)
    hx_pad = jnp.pad(hx, ((0, 0), (0, 0), (0, C - H)))
    src2d = jnp.concatenate(
        [data.astype(jnp.float32)[:, :, None, :], hx_pad[:, :, None, :]],
        axis=2).reshape(BS, 2 * N, C)

    b1r = jnp.transpose(b1f, (1, 0))
    w2T = jnp.transpose(w2t, (1, 0))
    b2r = jnp.transpose(b2c, (1, 0))
    wcT = jnp.transpose(wckc, (0, 2, 1))
    bcr = jnp.transpose(bcc, (1, 0))

    lg0 = jnp.dot(jnp.maximum(b1r, 0.0), w2T) + b2r
    lg0 = lg0 - jnp.max(lg0, axis=1, keepdims=True)
    e0 = jnp.exp(lg0)
    a0 = e0 / jnp.sum(e0, axis=1, keepdims=True)

    full2 = lambda b, i: (0, 0)
    out = pl.pallas_call(
        _fused_kernel,
        out_shape=jax.ShapeDtypeStruct((BS, N, C_out), jnp.float32),
        grid=(BS, N // tp),
        in_specs=[
            pl.BlockSpec((None, L, tp), lambda b, i: (b, 0, i),
                         memory_space=pltpu.SMEM),
            pl.BlockSpec((None, 2 * N, C), lambda b, i: (b, 0, 0)),
            pl.BlockSpec((None, tp, C), lambda b, i: (b, i, 0)),
            pl.BlockSpec((None, tp, H), lambda b, i: (b, i, 0)),
            pl.BlockSpec((1, K), full2),
            pl.BlockSpec((1, H), full2),
            pl.BlockSpec((H, K), full2),
            pl.BlockSpec((1, K), full2),
            pl.BlockSpec((K, C, C_out), lambda b, i: (0, 0, 0)),
            pl.BlockSpec((1, C_out), full2),
        ],
        out_specs=pl.BlockSpec((None, tp, C_out), lambda b, i: (b, i, 0)),
        scratch_shapes=[pltpu.VMEM((L * tp, C), jnp.float32),
                        pltpu.VMEM((L * tp, C), jnp.float32),
                        pltpu.VMEM((L * tp, K), jnp.float32)],
        compiler_params=pltpu.CompilerParams(
            dimension_semantics=("parallel", "arbitrary"),
            vmem_limit_bytes=_VMEM_LIMIT),
    )(idx2, src2d, data.astype(jnp.float32), hx, a0, b1r, w2T, b2r, wcT, bcr)

    aux = {"local_index": local_index, "dist": dist}
    return (xyz, out), aux


# speedup vs baseline: 1.1836x; 1.0049x over previous
import functools

import jax
import jax.numpy as jnp
from jax.experimental import pallas as pl
from jax.experimental.pallas import tpu as pltpu

_VMEM_LIMIT = 64 * 1024 * 1024
_L = 32
_TP = 256
_GU = 8


def _dist_knn_kernel(nsel, xr_ref, xt_ref, dist_ref, idx_ref):
    TI, N = dist_ref.shape
    d = jnp.zeros((TI, N), jnp.float32)
    for c in range(3):
        diff = xr_ref[:, c:c + 1] - xt_ref[c:c + 1, :]
        d = d + diff * diff
    d = jnp.where(d < 1e-8, jnp.inf, d)
    dist_ref[...] = d

    iota = jax.lax.broadcasted_iota(jnp.int32, (TI, N), 1)
    self_col = (jax.lax.broadcasted_iota(jnp.int32, (TI, 1), 0)
                + pl.program_id(1) * TI)
    cols = [self_col]
    w = d
    for _ in range(nsel):
        g = jnp.min(w, axis=1, keepdims=True)
        cand = jnp.where(w == g, iota, N)
        j = jnp.min(cand, axis=1, keepdims=True)
        cols.append(j)
        w = jnp.where(cand == j, jnp.inf, w)
    idx_ref[...] = jnp.concatenate(cols, axis=1)


def _masked_sqdist_knn(xyz, L):
    BS, N, _ = xyz.shape
    xyz = xyz.astype(jnp.float32)
    TI = 256 if N % 256 == 0 else N
    xyz_t = jnp.transpose(xyz, (0, 2, 1))
    return pl.pallas_call(
        functools.partial(_dist_knn_kernel, L - 1),
        out_shape=(jax.ShapeDtypeStruct((BS, N, N), jnp.float32),
                   jax.ShapeDtypeStruct((BS, N, L), jnp.int32)),
        grid=(BS, N // TI),
        in_specs=[pl.BlockSpec((None, TI, 3), lambda b, i: (b, i, 0)),
                  pl.BlockSpec((None, 3, N), lambda b, i: (b, 0, 0))],
        out_specs=(pl.BlockSpec((None, TI, N), lambda b, i: (b, i, 0)),
                   pl.BlockSpec((None, TI, L), lambda b, i: (b, i, 0))),
        compiler_params=pltpu.CompilerParams(
            dimension_semantics=("parallel", "arbitrary"),
            vmem_limit_bytes=_VMEM_LIMIT),
    )(xyz, xyz_t)


def _fused_kernel(idx_ref, src_ref, ds_ref, hs_ref, a0_ref, b1_ref, w2_ref,
                  b2_ref, wc_ref, bc_ref, out_ref, gd_ref, gh_ref, a_ref):
    L, tp = idx_ref.shape
    H = w2_ref.shape[0]
    K = w2_ref.shape[1]
    C_out = out_ref.shape[1]

    def gather_chunk(c, carry):
        for u in range(_GU):
            p = c * _GU + u
            for l in range(1, L):
                i2 = pl.multiple_of(idx_ref[l, p], 2)
                slab = src_ref[pl.ds(i2, 2), :]
                slot = l * tp + p
                gd_ref[pl.ds(slot, 1), :] = slab[0:1, :]
                gh_ref[pl.ds(slot, 1), :] = slab[1:2, :]
        return carry

    jax.lax.fori_loop(0, tp // _GU, gather_chunk, 0, unroll=False)

    b1 = b1_ref[...]
    b2 = b2_ref[...]
    h0 = hs_ref[...]

    for l in range(1, L):
        hg = gh_ref[l * tp:(l + 1) * tp, 0:H]
        h = jnp.maximum(hg - h0 + b1, 0.0)
        lg = jnp.dot(h, w2_ref[...],
                     preferred_element_type=jnp.float32) + b2
        lg = lg - jnp.max(lg, axis=1, keepdims=True)
        e = jnp.exp(lg)
        a_ref[l * tp:(l + 1) * tp, :] = e / jnp.sum(e, axis=1, keepdims=True)

    dself = ds_ref[...]
    out = jnp.broadcast_to(bc_ref[...], (tp, C_out))
    for k0 in range(0, K, 2):
        accs = [dself * a0_ref[0:1, k0 + t:k0 + t + 1] for t in range(2)]
        for l in range(1, L):
            d = gd_ref[l * tp:(l + 1) * tp, :]
            for t in range(2):
                w = a_ref[l * tp:(l + 1) * tp, k0 + t:k0 + t + 1]
                accs[t] = accs[t] + d * w
        for t in range(2):
            out = out + jnp.dot(accs[t], wc_ref[k0 + t],
                                preferred_element_type=jnp.float32)
    out_ref[...] = out


def kernel(xyz, data, w1f, b1f, w2t, b2c, wckc, bcc):
    BS, N, C = data.shape
    L = _L
    K, C_out = wckc.shape[0], wckc.shape[1]
    H = w1f.shape[0]
    tp = _TP if N % _TP == 0 else N

    dist, local_index = _masked_sqdist_knn(xyz, L)

    idx2 = jnp.transpose(local_index, (0, 2, 1)) * 2

    hx = jnp.einsum("bnd,hd->bnh", xyz.astype(jnp.float32), w1f)
    hx_pad = jnp.pad(hx, ((0, 0), (0, 0), (0, C - H)))
    src2d = jnp.concatenate(
        [data.astype(jnp.float32)[:, :, None, :], hx_pad[:, :, None, :]],
        axis=2).reshape(BS, 2 * N, C)

    b1r = jnp.transpose(b1f, (1, 0))
    w2T = jnp.transpose(w2t, (1, 0))
    b2r = jnp.transpose(b2c, (1, 0))
    wcT = jnp.transpose(wckc, (0, 2, 1))
    bcr = jnp.transpose(bcc, (1, 0))

    lg0 = jnp.dot(jnp.maximum(b1r, 0.0), w2T) + b2r
    lg0 = lg0 - jnp.max(lg0, axis=1, keepdims=True)
    e0 = jnp.exp(lg0)
    a0 = e0 / jnp.sum(e0, axis=1, keepdims=True)

    full2 = lambda b, i: (0, 0)
    out = pl.pallas_call(
        _fused_kernel,
        out_shape=jax.ShapeDtypeStruct((BS, N, C_out), jnp.float32),
        grid=(BS, N // tp),
        in_specs=[
            pl.BlockSpec((None, L, tp), lambda b, i: (b, 0, i),
                         memory_space=pltpu.SMEM),
            pl.BlockSpec((None, 2 * N, C), lambda b, i: (b, 0, 0)),
            pl.BlockSpec((None, tp, C), lambda b, i: (b, i, 0)),
            pl.BlockSpec((None, tp, H), lambda b, i: (b, i, 0)),
            pl.BlockSpec((1, K), full2),
            pl.BlockSpec((1, H), full2),
            pl.BlockSpec((H, K), full2),
            pl.BlockSpec((1, K), full2),
            pl.BlockSpec((K, C, C_out), lambda b, i: (0, 0, 0)),
            pl.BlockSpec((1, C_out), full2),
        ],
        out_specs=pl.BlockSpec((None, tp, C_out), lambda b, i: (b, i, 0)),
        scratch_shapes=[pltpu.VMEM((L * tp, C), jnp.float32),
                        pltpu.VMEM((L * tp, C), jnp.float32),
                        pltpu.VMEM((L * tp, K), jnp.float32)],
        compiler_params=pltpu.CompilerParams(
            dimension_semantics=("parallel", "arbitrary"),
            vmem_limit_bytes=_VMEM_LIMIT),
    )(idx2, src2d, data.astype(jnp.float32), hx, a0, b1r, w2T, b2r, wcT, bcr)

    aux = {"local_index": local_index, "dist": dist}
    return (xyz, out), aux
